# baseline (speedup 1.0000x reference)
import functools

import jax
import jax.numpy as jnp
from jax import lax
from jax.experimental import pallas as pl
from jax.experimental.pallas import tpu as pltpu
from jax.experimental.pallas import tpu_sc as plsc

N = 100000
C = 256
K = 64000
KHALF = K // 2
B = 80
NBLK = N // B
BS = 80
NBLKS = N // BS
NC, NS, L = 2, 16, 16
NW = NC * NS
CHW = 16
NPASS = C // CHW // NC
SLAB = K // NS
ZR = 100
CSLAB = KHALF // NS
TBPC = pl.cdiv(NBLKS, NS)
KPAD = 1280
WCH = 100
RB = 512
GRID = K // RB
EPSV = 1e-5


GB = 16
GROWS = GB * BS
NG = 5


def _sc_scatter_body(coords_hbm, feats_hbm, sums_hbm, counts_hbm, keys_hbm,
                     table, bb0, bb1, cbuf, obuf, zbuf, keys_v,
                     lsem, ssem, wsem):
    cid = lax.axis_index("c")
    tid = lax.axis_index("s")
    iota = lax.iota(jnp.int32, L)
    zero16 = jnp.zeros((L,), jnp.float32)
    one0 = jnp.where(iota == 0, 1.0, 0.0).astype(jnp.float32)

    nb = jnp.where(tid < 2, 79, 78)
    blk0 = tid * 78 + jnp.minimum(tid, 2)
    row0 = blk0 * BS

    def _init_row(r, _):
        zbuf[r, pl.ds(0, L)] = zero16
    lax.fori_loop(0, ZR, _init_row, None)

    def _ones_row(r, _):
        obuf[r, pl.ds(0, L)] = one0
    lax.fori_loop(0, BS, _ones_row, None)

    def _group_load(dst, src_hbm, q, col0, ncol, sem=None):
        r0 = row0 + q * GROWS

        def _cp(n):
            if ncol is None:
                src = src_hbm.at[pl.ds(r0, n), :]
            else:
                src = src_hbm.at[pl.ds(r0, n), pl.ds(col0, ncol)]
            if sem is None:
                pltpu.sync_copy(src, dst.at[pl.ds(0, n)])
            else:
                pltpu.async_copy(src, dst.at[pl.ds(0, n)], sem)
        if q < NG - 1:
            _cp(GROWS)
        else:
            @pl.when(tid < NS - 1)
            def _():
                _cp(15 * BS)
            @pl.when(tid == NS - 1)
            def _():
                _cp(14 * BS)

    def _group_load_wait(dst, q):
        def _wt(n):
            pltpu.make_async_copy(feats_hbm.at[pl.ds(0, n), pl.ds(0, CHW)],
                                  dst.at[pl.ds(0, n)], lsem).wait()
        if q < NG - 1:
            _wt(GROWS)
        else:
            @pl.when(tid < NS - 1)
            def _():
                _wt(15 * BS)
            @pl.when(tid == NS - 1)
            def _():
                _wt(14 * BS)

    for q in range(NG):
        _group_load(cbuf, coords_hbm, q, None, None)
        gcnt = jnp.minimum(nb - q * GB, GB)

        def _key_chunk(j, _):
            r = j * L + iota
            z = jnp.zeros((L,), jnp.int32)
            c0 = plsc.load_gather(cbuf, [r, z])
            c1 = plsc.load_gather(cbuf, [r, z + 1])
            c3 = plsc.load_gather(cbuf, [r, z + 3])
            keys_v[q * GB + j // 5, pl.ds((j % 5) * L, L)] = (
                c0 * 1600 + c1 * 40 + c3)
        lax.fori_loop(0, gcnt * (BS // L), _key_chunk, None)

    @pl.when(cid == 0)
    def _():
        pltpu.sync_copy(keys_v.at[pl.ds(0, 78)],
                        keys_hbm.at[pl.ds(blk0, 78), :])
        @pl.when(tid < 2)
        def _():
            pltpu.sync_copy(keys_v.at[78], keys_hbm.at[blk0 + 78])

    def _zero_slab():
        def _zf(j, _):
            pltpu.async_copy(zbuf, table.at[pl.ds(tid * SLAB + j * ZR, ZR)],
                             wsem)
        lax.fori_loop(0, SLAB // ZR, _zf, None)

        def _zd(j, _):
            pltpu.make_async_copy(
                zbuf, table.at[pl.ds(tid * SLAB, ZR)], wsem).wait()
        lax.fori_loop(0, SLAB // ZR, _zd, None)

    def _fire_scatters(q, src):
        gcnt = jnp.minimum(nb - q * GB, GB)

        def _fire(j, _):
            pltpu.async_copy(src.at[pl.ds(j * BS, BS)],
                             table.at[keys_v.at[q * GB + j]], ssem, add=True)
        lax.fori_loop(0, gcnt, _fire, None)
        return gcnt

    def _drain_scatters(gcnt, src):
        def _drain(j, _):
            pltpu.make_async_copy(src.at[pl.ds(0, BS)],
                                  table.at[keys_v.at[0]], ssem).wait()
        lax.fori_loop(0, gcnt, _drain, None)

    bbs = (bb0, bb1)

    def _pass(p, _):
        col0 = cid * (NPASS * CHW) + p * CHW
        _zero_slab()
        plsc.subcore_barrier()
        _group_load(bb0, feats_hbm, 0, col0, CHW)
        for q in range(NG):
            if q + 1 < NG:
                if q >= 1:
                    _drain_scatters(GB, bbs[(q + 1) % 2])
                _group_load(bbs[(q + 1) % 2], feats_hbm, q + 1, col0, CHW,
                            sem=lsem)
            _fire_scatters(q, bbs[q % 2])
            if q + 1 < NG:
                _group_load_wait(bbs[(q + 1) % 2], q + 1)
        _drain_scatters(nb - (NG - 2) * GB, bb0)
        plsc.subcore_barrier()

        def _wf(j, _):
            r0 = tid * SLAB + j * ZR
            pltpu.async_copy(table.at[pl.ds(r0, ZR)],
                             sums_hbm.at[pl.ds(r0, ZR), pl.ds(col0, CHW)],
                             wsem)
        lax.fori_loop(0, SLAB // ZR, _wf, None)

        def _wd(j, _):
            pltpu.make_async_copy(
                table.at[pl.ds(tid * SLAB, ZR)],
                sums_hbm.at[pl.ds(tid * SLAB, ZR), pl.ds(col0, CHW)],
                wsem).wait()
        lax.fori_loop(0, SLAB // ZR, _wd, None)
    lax.fori_loop(0, NPASS, _pass, None)

    _zero_slab()

    def _ckey(m, _):
        k16 = keys_v[m // 5, pl.ds((m % 5) * L, L)]
        lk = k16 - cid * KHALF
        ok = (lk >= 0) & (lk < KHALF)
        keys_v[m // 5, pl.ds((m % 5) * L, L)] = jnp.where(ok, lk, KHALF)
    lax.fori_loop(0, nb * (BS // L), _ckey, None)
    plsc.subcore_barrier()

    def _cfire(i, _):
        pltpu.async_copy(obuf, table.at[keys_v.at[i]], ssem, add=True)
    lax.fori_loop(0, nb, _cfire, None)

    def _cdrain(i, _):
        pltpu.make_async_copy(obuf, table.at[keys_v.at[0]], ssem).wait()
    lax.fori_loop(0, nb, _cdrain, None)
    plsc.subcore_barrier()
    pltpu.sync_copy(table.at[pl.ds(tid * CSLAB, CSLAB)],
                    counts_hbm.at[pl.ds(cid * KHALF + tid * CSLAB, CSLAB), :])


@functools.cache
def _sc_scatter():
    return pl.kernel(
        _sc_scatter_body,
        out_type=(jax.ShapeDtypeStruct((K, C), jnp.float32),
                  jax.ShapeDtypeStruct((K, CHW), jnp.float32),
                  jax.ShapeDtypeStruct((KPAD, BS), jnp.int32)),
        mesh=plsc.VectorSubcoreMesh(core_axis_name="c", subcore_axis_name="s",
                                    num_cores=NC, num_subcores=NS),
        compiler_params=pltpu.CompilerParams(use_tc_tiling_on_sc=False, needs_layout_passes=False),
        scratch_types=[
            pltpu.VMEM_SHARED((K, CHW), jnp.float32),
            pltpu.VMEM((GROWS, CHW), jnp.float32),
            pltpu.VMEM((GROWS, CHW), jnp.float32),
            pltpu.VMEM((GROWS, 4), jnp.int32),
            pltpu.VMEM((BS, CHW), jnp.float32),
            pltpu.VMEM((ZR, CHW), jnp.float32),
            pltpu.VMEM((TBPC, BS), jnp.int32),
            pltpu.SemaphoreType.DMA,
            pltpu.SemaphoreType.DMA,
            pltpu.SemaphoreType.DMA,
        ],
    )


def _p1_body(sums_ref, counts_ref, w1_ref, b1_ref, g1_ref, beta1_ref,
             y1_ref, scale_ref, shift_ref, s_acc, q_acc, m_acc):
    i = pl.program_id(0)

    @pl.when(i == 0)
    def _():
        s_acc[...] = jnp.zeros_like(s_acc)
        q_acc[...] = jnp.zeros_like(q_acc)
        m_acc[...] = jnp.zeros_like(m_acc)

    cnt = counts_ref[:, 0:1]
    valid = cnt > 0.0
    maskf = valid.astype(jnp.float32)
    inv = jnp.where(valid, 1.0 / jnp.maximum(cnt, 1.0), 0.0)
    x = (sums_ref[...] * inv).astype(jnp.bfloat16)
    y = lax.dot_general(x, w1_ref[...].astype(jnp.bfloat16),
                        (((1,), (1,)), ((), ())),
                        preferred_element_type=jnp.float32) + b1_ref[...]
    y1_ref[...] = y
    ym = y * maskf
    s_acc[...] += jnp.sum(ym, axis=0, keepdims=True)
    q_acc[...] += jnp.sum(y * ym, axis=0, keepdims=True)
    m_acc[...] += jnp.sum(maskf).reshape(1, 1)

    @pl.when(i == GRID - 1)
    def _():
        m = m_acc[0, 0]
        mu = s_acc[...] / m
        var = q_acc[...] / m - mu * mu
        sc = g1_ref[...] * lax.rsqrt(var + EPSV)
        scale_ref[...] = sc
        shift_ref[...] = beta1_ref[...] - mu * sc


def _tc_p1(sums, counts, W1, b1, g1, beta1):
    return pl.pallas_call(
        _p1_body,
        grid=(GRID,),
        in_specs=[
            pl.BlockSpec((RB, C), lambda i: (i, 0)),
            pl.BlockSpec((RB, CHW), lambda i: (i, 0)),
            pl.BlockSpec((C, C), lambda i: (0, 0)),
            pl.BlockSpec((1, C), lambda i: (0, 0)),
            pl.BlockSpec((1, C), lambda i: (0, 0)),
            pl.BlockSpec((1, C), lambda i: (0, 0)),
        ],
        out_specs=[
            pl.BlockSpec((RB, C), lambda i: (i, 0)),
            pl.BlockSpec((1, C), lambda i: (0, 0)),
            pl.BlockSpec((1, C), lambda i: (0, 0)),
        ],
        out_shape=[
            jax.ShapeDtypeStruct((K, C), jnp.float32),
            jax.ShapeDtypeStruct((1, C), jnp.float32),
            jax.ShapeDtypeStruct((1, C), jnp.float32),
        ],
        scratch_shapes=[
            pltpu.VMEM((1, C), jnp.float32),
            pltpu.VMEM((1, C), jnp.float32),
            pltpu.VMEM((1, 1), jnp.float32),
        ],
    )(sums, counts, W1, b1.reshape(1, C), g1.reshape(1, C),
      beta1.reshape(1, C))


def _p2_body(y1_ref, counts_ref, sc1_ref, sh1_ref, w2_ref, b2_ref, g2_ref,
             beta2_ref, y2_ref, scale_ref, shift_ref, s_acc, q_acc, m_acc):
    i = pl.program_id(0)

    @pl.when(i == 0)
    def _():
        s_acc[...] = jnp.zeros_like(s_acc)
        q_acc[...] = jnp.zeros_like(q_acc)
        m_acc[...] = jnp.zeros_like(m_acc)

    cnt = counts_ref[:, 0:1]
    maskf = (cnt > 0.0).astype(jnp.float32)
    h = jnp.maximum(y1_ref[...] * sc1_ref[...] + sh1_ref[...], 0.0)
    y = lax.dot_general(h.astype(jnp.bfloat16),
                        w2_ref[...].astype(jnp.bfloat16),
                        (((1,), (1,)), ((), ())),
                        preferred_element_type=jnp.float32) + b2_ref[...]
    y2_ref[...] = y
    ym = y * maskf
    s_acc[...] += jnp.sum(ym, axis=0, keepdims=True)
    q_acc[...] += jnp.sum(y * ym, axis=0, keepdims=True)
    m_acc[...] += jnp.sum(maskf).reshape(1, 1)

    @pl.when(i == GRID - 1)
    def _():
        m = m_acc[0, 0]
        mu = s_acc[...] / m
        var = q_acc[...] / m - mu * mu
        sc = g2_ref[...] * lax.rsqrt(var + EPSV)
        scale_ref[...] = sc
        shift_ref[...] = beta2_ref[...] - mu * sc


def _tc_p2(y1, counts, sc1, sh1, W2, b2, g2, beta2):
    return pl.pallas_call(
        _p2_body,
        grid=(GRID,),
        in_specs=[
            pl.BlockSpec((RB, C), lambda i: (i, 0)),
            pl.BlockSpec((RB, CHW), lambda i: (i, 0)),
            pl.BlockSpec((1, C), lambda i: (0, 0)),
            pl.BlockSpec((1, C), lambda i: (0, 0)),
            pl.BlockSpec((C, C), lambda i: (0, 0)),
            pl.BlockSpec((1, C), lambda i: (0, 0)),
            pl.BlockSpec((1, C), lambda i: (0, 0)),
            pl.BlockSpec((1, C), lambda i: (0, 0)),
        ],
        out_specs=[
            pl.BlockSpec((RB, C), lambda i: (i, 0)),
            pl.BlockSpec((1, C), lambda i: (0, 0)),
            pl.BlockSpec((1, C), lambda i: (0, 0)),
        ],
        out_shape=[
            jax.ShapeDtypeStruct((K, C), jnp.float32),
            jax.ShapeDtypeStruct((1, C), jnp.float32),
            jax.ShapeDtypeStruct((1, C), jnp.float32),
        ],
        scratch_shapes=[
            pltpu.VMEM((1, C), jnp.float32),
            pltpu.VMEM((1, C), jnp.float32),
            pltpu.VMEM((1, 1), jnp.float32),
        ],
    )(y1, counts, sc1, sh1, W2, b2.reshape(1, C), g2.reshape(1, C),
      beta2.reshape(1, C))


GWB = 40


def _sc_gather_body(keys_hbm, feats_hbm, y2_hbm, sc_hbm, sh_hbm, out_hbm,
                    kbuf, fb0, fb1, rw0, rw1, scale_v, shift_v,
                    fsem, gsem, wsem):
    cid = lax.axis_index("c")
    sid = lax.axis_index("s")
    w = sid * NC + cid
    blk0 = w * GWB
    nb = jnp.minimum(NBLK - blk0, GWB)
    pltpu.sync_copy(sc_hbm, scale_v)
    pltpu.sync_copy(sh_hbm, shift_v)
    pltpu.sync_copy(keys_hbm.at[pl.ds(blk0, GWB), :], kbuf)
    svals = [scale_v[pl.ds(j * L, L)] for j in range(C // L)]
    tvals = [shift_v[pl.ds(j * L, L)] for j in range(C // L)]
    fbs, rws = (fb0, fb1), (rw0, rw1)

    def _feats_cp(i, buf):
        return pltpu.make_async_copy(
            feats_hbm.at[pl.ds((blk0 + i) * B, B), :], buf, fsem)

    def _gath_cp(i, buf):
        return pltpu.make_async_copy(y2_hbm.at[kbuf.at[i]], buf, gsem)

    def _out_cp(i, buf):
        return pltpu.make_async_copy(
            buf, out_hbm.at[pl.ds((blk0 + i) * B, B), :], wsem)

    _feats_cp(0, fb0).start()
    _gath_cp(0, rw0).start()

    def _outer(io, _):
        for par in range(2):
            i = io * 2 + par
            fb, rw = fbs[par], rws[par]
            fbn, rwn = fbs[1 - par], rws[1 - par]

            @pl.when(i < nb)
            def _():
                _feats_cp(i, fb).wait()
                _gath_cp(i, rw).wait()

                @pl.when(i + 1 < nb)
                def _():
                    @pl.when(i >= 1)
                    def _():
                        _out_cp(i - 1, fbn).wait()
                    _feats_cp(i + 1, fbn).start()
                    _gath_cp(i + 1, rwn).start()

                def _row(r, _):
                    for j in range(C // L):
                        y = rw[r, pl.ds(j * L, L)]
                        h = jnp.maximum(y * svals[j] + tvals[j], 0.0)
                        fb[r, pl.ds(j * L, L)] += h
                lax.fori_loop(0, B, _row, None)
                _out_cp(i, fb).start()
    lax.fori_loop(0, GWB // 2, _outer, None)
    _out_cp(nb - 2, fb0).wait()
    _out_cp(nb - 1, fb1).wait()


@functools.cache
def _sc_gather():
    return pl.kernel(
        _sc_gather_body,
        out_type=jax.ShapeDtypeStruct((N, C), jnp.float32),
        mesh=plsc.VectorSubcoreMesh(core_axis_name="c", subcore_axis_name="s",
                                    num_cores=NC, num_subcores=NS),
        scratch_types=[
            pltpu.VMEM((GWB, BS), jnp.int32),
            pltpu.VMEM((B, C), jnp.float32),
            pltpu.VMEM((B, C), jnp.float32),
            pltpu.VMEM((B, C), jnp.float32),
            pltpu.VMEM((B, C), jnp.float32),
            pltpu.VMEM((C,), jnp.float32),
            pltpu.VMEM((C,), jnp.float32),
            pltpu.SemaphoreType.DMA,
            pltpu.SemaphoreType.DMA,
            pltpu.SemaphoreType.DMA,
        ],
    )


def kernel(coords, feats, W1, b1, g1, beta1, W2, b2, g2, beta2):
    sums, counts, keys = _sc_scatter()(coords, feats)
    y1, sc1, sh1 = _tc_p1(sums, counts, W1, b1, g1, beta1)
    y2, sc2, sh2 = _tc_p2(y1, counts, sc1, sh1, W2, b2, g2, beta2)
    return _sc_gather()(keys, feats, y2, sc2.reshape(C), sh2.reshape(C))


# speedup vs baseline: 1.0143x; 1.0143x over previous
import functools

import jax
import jax.numpy as jnp
from jax import lax
from jax.experimental import pallas as pl
from jax.experimental.pallas import tpu as pltpu
from jax.experimental.pallas import tpu_sc as plsc

N = 100000
C = 256
K = 64000
KHALF = K // 2
B = 80
NBLK = N // B
BS = 80
NBLKS = N // BS
NC, NS, L = 2, 16, 16
NW = NC * NS
CHW = 16
NPASS = C // CHW // NC
SLAB = K // NS
ZR = 100
CSLAB = KHALF // NS
TBPC = pl.cdiv(NBLKS, NS)
KPAD = 1280
WCH = 100
RB = 512
GRID = K // RB
EPSV = 1e-5


GB = 16
GROWS = GB * BS
NG = 5


def _sc_scatter_body(coords_hbm, feats_hbm, sums_hbm, counts_hbm, keys_hbm,
                     table, bb0, bb1, cbuf, obuf, zbuf, keys_v,
                     lsem, ssem, wsem):
    cid = lax.axis_index("c")
    tid = lax.axis_index("s")
    iota = lax.iota(jnp.int32, L)
    zero16 = jnp.zeros((L,), jnp.float32)
    one0 = jnp.where(iota == 0, 1.0, 0.0).astype(jnp.float32)

    nb = jnp.where(tid < 2, 79, 78)
    blk0 = tid * 78 + jnp.minimum(tid, 2)
    row0 = blk0 * BS

    def _init_row(r, _):
        zbuf[r, pl.ds(0, L)] = zero16
    lax.fori_loop(0, ZR, _init_row, None)

    def _ones_row(r, _):
        obuf[r, pl.ds(0, L)] = one0
    lax.fori_loop(0, BS, _ones_row, None)

    def _group_load(dst, src_hbm, q, col0, ncol, sem=None):
        r0 = row0 + q * GROWS

        def _cp(n):
            if ncol is None:
                src = src_hbm.at[pl.ds(r0, n), :]
            else:
                src = src_hbm.at[pl.ds(r0, n), pl.ds(col0, ncol)]
            if sem is None:
                pltpu.sync_copy(src, dst.at[pl.ds(0, n)])
            else:
                pltpu.async_copy(src, dst.at[pl.ds(0, n)], sem)
        if q < NG - 1:
            _cp(GROWS)
        else:
            @pl.when(tid < NS - 1)
            def _():
                _cp(15 * BS)
            @pl.when(tid == NS - 1)
            def _():
                _cp(14 * BS)

    def _group_load_wait(dst, q):
        def _wt(n):
            pltpu.make_async_copy(feats_hbm.at[pl.ds(0, n), pl.ds(0, CHW)],
                                  dst.at[pl.ds(0, n)], lsem).wait()
        if q < NG - 1:
            _wt(GROWS)
        else:
            @pl.when(tid < NS - 1)
            def _():
                _wt(15 * BS)
            @pl.when(tid == NS - 1)
            def _():
                _wt(14 * BS)

    for q in range(NG):
        _group_load(cbuf, coords_hbm, q, None, None)
        gcnt = jnp.minimum(nb - q * GB, GB)

        def _key_chunk(j, _):
            r = j * L + iota
            z = jnp.zeros((L,), jnp.int32)
            c0 = plsc.load_gather(cbuf, [r, z])
            c1 = plsc.load_gather(cbuf, [r, z + 1])
            c3 = plsc.load_gather(cbuf, [r, z + 3])
            keys_v[q * GB + j // 5, pl.ds((j % 5) * L, L)] = (
                c0 * 1600 + c1 * 40 + c3)
        lax.fori_loop(0, gcnt * (BS // L), _key_chunk, None)

    @pl.when(cid == 0)
    def _():
        pltpu.sync_copy(keys_v.at[pl.ds(0, 78)],
                        keys_hbm.at[pl.ds(blk0, 78), :])
        @pl.when(tid < 2)
        def _():
            pltpu.sync_copy(keys_v.at[78], keys_hbm.at[blk0 + 78])

    def _zero_slab():
        def _zf(j, _):
            pltpu.async_copy(zbuf, table.at[pl.ds(tid * SLAB + j * ZR, ZR)],
                             wsem)
        lax.fori_loop(0, SLAB // ZR, _zf, None)

        def _zd(j, _):
            pltpu.make_async_copy(
                zbuf, table.at[pl.ds(tid * SLAB, ZR)], wsem).wait()
        lax.fori_loop(0, SLAB // ZR, _zd, None)

    def _fire_scatters(q, src):
        gcnt = jnp.minimum(nb - q * GB, GB)

        def _fire(j, _):
            pltpu.async_copy(src.at[pl.ds(j * BS, BS)],
                             table.at[keys_v.at[q * GB + j]], ssem, add=True)
        lax.fori_loop(0, gcnt, _fire, None)
        return gcnt

    def _drain_scatters(gcnt, src):
        def _drain(j, _):
            pltpu.make_async_copy(src.at[pl.ds(0, BS)],
                                  table.at[keys_v.at[0]], ssem).wait()
        lax.fori_loop(0, gcnt, _drain, None)

    bbs = (bb0, bb1)

    def _pass(p, _):
        col0 = cid * (NPASS * CHW) + p * CHW
        _zero_slab()
        plsc.subcore_barrier()
        _group_load(bb0, feats_hbm, 0, col0, CHW)
        for q in range(NG):
            if q + 1 < NG:
                if q >= 1:
                    _drain_scatters(GB, bbs[(q + 1) % 2])
                _group_load(bbs[(q + 1) % 2], feats_hbm, q + 1, col0, CHW,
                            sem=lsem)
            _fire_scatters(q, bbs[q % 2])
            if q + 1 < NG:
                _group_load_wait(bbs[(q + 1) % 2], q + 1)
        _drain_scatters(nb - (NG - 2) * GB, bb0)
        plsc.subcore_barrier()

        def _wf(j, _):
            r0 = tid * SLAB + j * ZR
            pltpu.async_copy(table.at[pl.ds(r0, ZR)],
                             sums_hbm.at[pl.ds(r0, ZR), pl.ds(col0, CHW)],
                             wsem)
        lax.fori_loop(0, SLAB // ZR, _wf, None)

        def _wd(j, _):
            pltpu.make_async_copy(
                table.at[pl.ds(tid * SLAB, ZR)],
                sums_hbm.at[pl.ds(tid * SLAB, ZR), pl.ds(col0, CHW)],
                wsem).wait()
        lax.fori_loop(0, SLAB // ZR, _wd, None)
    lax.fori_loop(0, NPASS, _pass, None)

    _zero_slab()

    def _ckey(m, _):
        k16 = keys_v[m // 5, pl.ds((m % 5) * L, L)]
        lk = k16 - cid * KHALF
        ok = (lk >= 0) & (lk < KHALF)
        keys_v[m // 5, pl.ds((m % 5) * L, L)] = jnp.where(ok, lk, KHALF)
    lax.fori_loop(0, nb * (BS // L), _ckey, None)
    plsc.subcore_barrier()

    def _cfire(i, _):
        pltpu.async_copy(obuf, table.at[keys_v.at[i]], ssem, add=True)
    lax.fori_loop(0, nb, _cfire, None)

    def _cdrain(i, _):
        pltpu.make_async_copy(obuf, table.at[keys_v.at[0]], ssem).wait()
    lax.fori_loop(0, nb, _cdrain, None)
    plsc.subcore_barrier()
    pltpu.sync_copy(table.at[pl.ds(tid * CSLAB, CSLAB)],
                    counts_hbm.at[pl.ds(cid * KHALF + tid * CSLAB, CSLAB), :])


@functools.cache
def _sc_scatter():
    return pl.kernel(
        _sc_scatter_body,
        out_type=(jax.ShapeDtypeStruct((K, C), jnp.float32),
                  jax.ShapeDtypeStruct((K, CHW), jnp.float32),
                  jax.ShapeDtypeStruct((KPAD, BS), jnp.int32)),
        mesh=plsc.VectorSubcoreMesh(core_axis_name="c", subcore_axis_name="s",
                                    num_cores=NC, num_subcores=NS),
        compiler_params=pltpu.CompilerParams(use_tc_tiling_on_sc=False, needs_layout_passes=False),
        scratch_types=[
            pltpu.VMEM_SHARED((K, CHW), jnp.float32),
            pltpu.VMEM((GROWS, CHW), jnp.float32),
            pltpu.VMEM((GROWS, CHW), jnp.float32),
            pltpu.VMEM((GROWS, 4), jnp.int32),
            pltpu.VMEM((BS, CHW), jnp.float32),
            pltpu.VMEM((ZR, CHW), jnp.float32),
            pltpu.VMEM((TBPC, BS), jnp.int32),
            pltpu.SemaphoreType.DMA,
            pltpu.SemaphoreType.DMA,
            pltpu.SemaphoreType.DMA,
        ],
    )


def _p1_body(sums_ref, counts_ref, w1_ref, b1_ref, g1_ref, beta1_ref,
             y1_ref, scale_ref, shift_ref, s_acc, q_acc, m_acc):
    i = pl.program_id(0)

    @pl.when(i == 0)
    def _():
        s_acc[...] = jnp.zeros_like(s_acc)
        q_acc[...] = jnp.zeros_like(q_acc)
        m_acc[...] = jnp.zeros_like(m_acc)

    cnt = counts_ref[:, 0:1]
    valid = cnt > 0.0
    maskf = valid.astype(jnp.float32)
    inv = jnp.where(valid, 1.0 / jnp.maximum(cnt, 1.0), 0.0)
    x = (sums_ref[...] * inv).astype(jnp.bfloat16)
    y = lax.dot_general(x, w1_ref[...].astype(jnp.bfloat16),
                        (((1,), (1,)), ((), ())),
                        preferred_element_type=jnp.float32) + b1_ref[...]
    y1_ref[...] = y.astype(jnp.bfloat16)
    ym = y * maskf
    s_acc[...] += jnp.sum(ym, axis=0, keepdims=True)
    q_acc[...] += jnp.sum(y * ym, axis=0, keepdims=True)
    m_acc[...] += jnp.sum(maskf).reshape(1, 1)

    @pl.when(i == GRID - 1)
    def _():
        m = m_acc[0, 0]
        mu = s_acc[...] / m
        var = q_acc[...] / m - mu * mu
        sc = g1_ref[...] * lax.rsqrt(var + EPSV)
        scale_ref[...] = sc
        shift_ref[...] = beta1_ref[...] - mu * sc


def _tc_p1(sums, counts, W1, b1, g1, beta1):
    return pl.pallas_call(
        _p1_body,
        grid=(GRID,),
        in_specs=[
            pl.BlockSpec((RB, C), lambda i: (i, 0)),
            pl.BlockSpec((RB, CHW), lambda i: (i, 0)),
            pl.BlockSpec((C, C), lambda i: (0, 0)),
            pl.BlockSpec((1, C), lambda i: (0, 0)),
            pl.BlockSpec((1, C), lambda i: (0, 0)),
            pl.BlockSpec((1, C), lambda i: (0, 0)),
        ],
        out_specs=[
            pl.BlockSpec((RB, C), lambda i: (i, 0)),
            pl.BlockSpec((1, C), lambda i: (0, 0)),
            pl.BlockSpec((1, C), lambda i: (0, 0)),
        ],
        out_shape=[
            jax.ShapeDtypeStruct((K, C), jnp.bfloat16),
            jax.ShapeDtypeStruct((1, C), jnp.float32),
            jax.ShapeDtypeStruct((1, C), jnp.float32),
        ],
        scratch_shapes=[
            pltpu.VMEM((1, C), jnp.float32),
            pltpu.VMEM((1, C), jnp.float32),
            pltpu.VMEM((1, 1), jnp.float32),
        ],
    )(sums, counts, W1, b1.reshape(1, C), g1.reshape(1, C),
      beta1.reshape(1, C))


def _p2_body(y1_ref, counts_ref, sc1_ref, sh1_ref, w2_ref, b2_ref, g2_ref,
             beta2_ref, y2_ref, scale_ref, shift_ref, s_acc, q_acc, m_acc):
    i = pl.program_id(0)

    @pl.when(i == 0)
    def _():
        s_acc[...] = jnp.zeros_like(s_acc)
        q_acc[...] = jnp.zeros_like(q_acc)
        m_acc[...] = jnp.zeros_like(m_acc)

    cnt = counts_ref[:, 0:1]
    maskf = (cnt > 0.0).astype(jnp.float32)
    h = jnp.maximum(y1_ref[...].astype(jnp.float32) * sc1_ref[...]
                    + sh1_ref[...], 0.0)
    y = lax.dot_general(h.astype(jnp.bfloat16),
                        w2_ref[...].astype(jnp.bfloat16),
                        (((1,), (1,)), ((), ())),
                        preferred_element_type=jnp.float32) + b2_ref[...]
    y2_ref[...] = y
    ym = y * maskf
    s_acc[...] += jnp.sum(ym, axis=0, keepdims=True)
    q_acc[...] += jnp.sum(y * ym, axis=0, keepdims=True)
    m_acc[...] += jnp.sum(maskf).reshape(1, 1)

    @pl.when(i == GRID - 1)
    def _():
        m = m_acc[0, 0]
        mu = s_acc[...] / m
        var = q_acc[...] / m - mu * mu
        sc = g2_ref[...] * lax.rsqrt(var + EPSV)
        scale_ref[...] = sc
        shift_ref[...] = beta2_ref[...] - mu * sc


def _tc_p2(y1, counts, sc1, sh1, W2, b2, g2, beta2):
    return pl.pallas_call(
        _p2_body,
        grid=(GRID,),
        in_specs=[
            pl.BlockSpec((RB, C), lambda i: (i, 0)),
            pl.BlockSpec((RB, CHW), lambda i: (i, 0)),
            pl.BlockSpec((1, C), lambda i: (0, 0)),
            pl.BlockSpec((1, C), lambda i: (0, 0)),
            pl.BlockSpec((C, C), lambda i: (0, 0)),
            pl.BlockSpec((1, C), lambda i: (0, 0)),
            pl.BlockSpec((1, C), lambda i: (0, 0)),
            pl.BlockSpec((1, C), lambda i: (0, 0)),
        ],
        out_specs=[
            pl.BlockSpec((RB, C), lambda i: (i, 0)),
            pl.BlockSpec((1, C), lambda i: (0, 0)),
            pl.BlockSpec((1, C), lambda i: (0, 0)),
        ],
        out_shape=[
            jax.ShapeDtypeStruct((K, C), jnp.float32),
            jax.ShapeDtypeStruct((1, C), jnp.float32),
            jax.ShapeDtypeStruct((1, C), jnp.float32),
        ],
        scratch_shapes=[
            pltpu.VMEM((1, C), jnp.float32),
            pltpu.VMEM((1, C), jnp.float32),
            pltpu.VMEM((1, 1), jnp.float32),
        ],
    )(y1, counts, sc1, sh1, W2, b2.reshape(1, C), g2.reshape(1, C),
      beta2.reshape(1, C))


GWB = 40


def _sc_gather_body(keys_hbm, feats_hbm, y2_hbm, sc_hbm, sh_hbm, out_hbm,
                    kbuf, fb0, fb1, rw0, rw1, scale_v, shift_v,
                    fsem, gsem, wsem):
    cid = lax.axis_index("c")
    sid = lax.axis_index("s")
    w = sid * NC + cid
    blk0 = w * GWB
    nb = jnp.minimum(NBLK - blk0, GWB)
    pltpu.sync_copy(sc_hbm, scale_v)
    pltpu.sync_copy(sh_hbm, shift_v)
    pltpu.sync_copy(keys_hbm.at[pl.ds(blk0, GWB), :], kbuf)
    svals = [scale_v[pl.ds(j * L, L)] for j in range(C // L)]
    tvals = [shift_v[pl.ds(j * L, L)] for j in range(C // L)]
    fbs, rws = (fb0, fb1), (rw0, rw1)

    def _feats_cp(i, buf):
        return pltpu.make_async_copy(
            feats_hbm.at[pl.ds((blk0 + i) * B, B), :], buf, fsem)

    def _gath_cp(i, buf):
        return pltpu.make_async_copy(y2_hbm.at[kbuf.at[i]], buf, gsem)

    def _out_cp(i, buf):
        return pltpu.make_async_copy(
            buf, out_hbm.at[pl.ds((blk0 + i) * B, B), :], wsem)

    _feats_cp(0, fb0).start()
    _gath_cp(0, rw0).start()

    def _outer(io, _):
        for par in range(2):
            i = io * 2 + par
            fb, rw = fbs[par], rws[par]
            fbn, rwn = fbs[1 - par], rws[1 - par]

            @pl.when(i < nb)
            def _():
                _feats_cp(i, fb).wait()
                _gath_cp(i, rw).wait()

                @pl.when(i + 1 < nb)
                def _():
                    @pl.when(i >= 1)
                    def _():
                        _out_cp(i - 1, fbn).wait()
                    _feats_cp(i + 1, fbn).start()
                    _gath_cp(i + 1, rwn).start()

                def _row(r, _):
                    for j in range(C // L):
                        y = rw[r, pl.ds(j * L, L)]
                        h = jnp.maximum(y * svals[j] + tvals[j], 0.0)
                        fb[r, pl.ds(j * L, L)] += h
                lax.fori_loop(0, B, _row, None)
                _out_cp(i, fb).start()
    lax.fori_loop(0, GWB // 2, _outer, None)
    _out_cp(nb - 2, fb0).wait()
    _out_cp(nb - 1, fb1).wait()


@functools.cache
def _sc_gather():
    return pl.kernel(
        _sc_gather_body,
        out_type=jax.ShapeDtypeStruct((N, C), jnp.float32),
        mesh=plsc.VectorSubcoreMesh(core_axis_name="c", subcore_axis_name="s",
                                    num_cores=NC, num_subcores=NS),
        scratch_types=[
            pltpu.VMEM((GWB, BS), jnp.int32),
            pltpu.VMEM((B, C), jnp.float32),
            pltpu.VMEM((B, C), jnp.float32),
            pltpu.VMEM((B, C), jnp.float32),
            pltpu.VMEM((B, C), jnp.float32),
            pltpu.VMEM((C,), jnp.float32),
            pltpu.VMEM((C,), jnp.float32),
            pltpu.SemaphoreType.DMA,
            pltpu.SemaphoreType.DMA,
            pltpu.SemaphoreType.DMA,
        ],
    )


def kernel(coords, feats, W1, b1, g1, beta1, W2, b2, g2, beta2):
    sums, counts, keys = _sc_scatter()(coords, feats)
    y1, sc1, sh1 = _tc_p1(sums, counts, W1, b1, g1, beta1)
    y2, sc2, sh2 = _tc_p2(y1, counts, sc1, sh1, W2, b2, g2, beta2)
    return _sc_gather()(keys, feats, y2, sc2.reshape(C), sh2.reshape(C))


# speedup vs baseline: 1.0691x; 1.0540x over previous
import functools

import jax
import jax.numpy as jnp
from jax import lax
from jax.experimental import pallas as pl
from jax.experimental.pallas import tpu as pltpu
from jax.experimental.pallas import tpu_sc as plsc

N = 100000
C = 256
K = 64000
KHALF = K // 2
B = 80
NBLK = N // B
BS = 80
NBLKS = N // BS
NC, NS, L = 2, 16, 16
NW = NC * NS
CHW = 32
NPASS = C // CHW // NC
SLAB = K // NS
ZR = 125
CSLAB = KHALF // NS
TBPC = pl.cdiv(NBLKS, NS)
KPAD = 1280
WCH = 100
RB = 512
GRID = K // RB
EPSV = 1e-5


GB = 8
GROWS = GB * BS
NG = 10
TAILA = 79 - (NG - 1) * GB
TAILB = 78 - (NG - 1) * GB


def _sc_scatter_body(coords_hbm, feats_hbm, sums_hbm, counts_hbm, keys_hbm,
                     table, bf0, bf1, bbh, cbuf, obuf, zbuf, keys_v,
                     lsem, ssem, wsem):
    cid = lax.axis_index("c")
    tid = lax.axis_index("s")
    iota = lax.iota(jnp.int32, L)
    zero16 = jnp.zeros((L,), jnp.float32)
    one0 = jnp.where(iota == 0, 1.0, 0.0).astype(jnp.float32)
    zero32h = plsc.pack(zero16, zero16, format=plsc.PackFormat.INTERLEAVED)
    one0h = plsc.pack(one0, zero16, format=plsc.PackFormat.INTERLEAVED)

    nb = jnp.where(tid < 2, 79, 78)
    blk0 = tid * 78 + jnp.minimum(tid, 2)
    row0 = blk0 * BS

    def _init_row(r, _):
        zbuf[r, :] = zero32h
    lax.fori_loop(0, ZR, _init_row, None)

    def _ones_row(r, _):
        obuf[r, :] = one0h
    lax.fori_loop(0, BS, _ones_row, None)

    def _group_load(dst, src_hbm, q, col0, ncol, sem=None):
        r0 = row0 + q * GROWS

        def _cp(n):
            if ncol is None:
                src = src_hbm.at[pl.ds(r0, n), :]
            else:
                src = src_hbm.at[pl.ds(r0, n), pl.ds(col0, ncol)]
            if sem is None:
                pltpu.sync_copy(src, dst.at[pl.ds(0, n)])
            else:
                pltpu.async_copy(src, dst.at[pl.ds(0, n)], sem)
        if q < NG - 1:
            _cp(GROWS)
        else:
            @pl.when(tid < NS - 1)
            def _():
                _cp(TAILA * BS)
            @pl.when(tid == NS - 1)
            def _():
                _cp(TAILB * BS)

    def _group_load_wait(dst, q):
        def _wt(n):
            pltpu.make_async_copy(feats_hbm.at[pl.ds(0, n), pl.ds(0, CHW)],
                                  dst.at[pl.ds(0, n)], lsem).wait()
        if q < NG - 1:
            _wt(GROWS)
        else:
            @pl.when(tid < NS - 1)
            def _():
                _wt(TAILA * BS)
            @pl.when(tid == NS - 1)
            def _():
                _wt(TAILB * BS)

    for q in range(NG):
        _group_load(cbuf, coords_hbm, q, None, None)
        gcnt = jnp.minimum(nb - q * GB, GB)

        def _key_chunk(j, _):
            r = j * L + iota
            z = jnp.zeros((L,), jnp.int32)
            c0 = plsc.load_gather(cbuf, [r, z])
            c1 = plsc.load_gather(cbuf, [r, z + 1])
            c3 = plsc.load_gather(cbuf, [r, z + 3])
            keys_v[q * GB + j // 5, pl.ds((j % 5) * L, L)] = (
                c0 * 1600 + c1 * 40 + c3)
        lax.fori_loop(0, gcnt * (BS // L), _key_chunk, None)

    @pl.when(cid == 0)
    def _():
        pltpu.sync_copy(keys_v.at[pl.ds(0, 78)],
                        keys_hbm.at[pl.ds(blk0, 78), :])
        @pl.when(tid < 2)
        def _():
            pltpu.sync_copy(keys_v.at[78], keys_hbm.at[blk0 + 78])

    def _zero_slab():
        def _zf(j, _):
            pltpu.async_copy(zbuf, table.at[pl.ds(tid * SLAB + j * ZR, ZR)],
                             wsem)
        lax.fori_loop(0, SLAB // ZR, _zf, None)

        def _zd(j, _):
            pltpu.make_async_copy(
                zbuf, table.at[pl.ds(tid * SLAB, ZR)], wsem).wait()
        lax.fori_loop(0, SLAB // ZR, _zd, None)

    def _fire_scatters(q, src):
        gcnt = jnp.minimum(nb - q * GB, GB)

        def _fire(j, _):
            pltpu.async_copy(src.at[pl.ds(j * BS, BS)],
                             table.at[keys_v.at[q * GB + j]], ssem, add=True)
        lax.fori_loop(0, gcnt, _fire, None)
        return gcnt

    def _drain_scatters(gcnt, src):
        def _drain(j, _):
            pltpu.make_async_copy(src.at[pl.ds(0, BS)],
                                  table.at[keys_v.at[0]], ssem).wait()
        lax.fori_loop(0, gcnt, _drain, None)

    ev_idx = iota * 2
    od_idx = iota * 2 + 1
    bfs = (bf0, bf1)

    def _convert(q, src):
        gcnt = jnp.minimum(nb - q * GB, GB)

        def _cv(r, _):
            rr = jnp.zeros((L,), jnp.int32) + r
            a = plsc.load_gather(src, [rr, ev_idx])
            b = plsc.load_gather(src, [rr, od_idx])
            bbh[r, :] = plsc.pack(a, b, format=plsc.PackFormat.INTERLEAVED)
        lax.fori_loop(0, gcnt * BS, _cv, None)

    def _pass(p, _):
        col0 = cid * (NPASS * CHW) + p * CHW
        _zero_slab()
        plsc.subcore_barrier()
        _group_load(bf0, feats_hbm, 0, col0, CHW)
        for q in range(NG):
            if q + 1 < NG:
                _group_load(bfs[(q + 1) % 2], feats_hbm, q + 1, col0, CHW,
                            sem=lsem)
            if q >= 1:
                _drain_scatters(GB, bbh)
            _convert(q, bfs[q % 2])
            _fire_scatters(q, bbh)
            if q + 1 < NG:
                _group_load_wait(bfs[(q + 1) % 2], q + 1)
        _drain_scatters(nb - (NG - 1) * GB, bbh)
        plsc.subcore_barrier()

        def _wf(j, _):
            r0 = tid * SLAB + j * ZR
            pltpu.async_copy(table.at[pl.ds(r0, ZR)],
                             sums_hbm.at[pl.ds(r0, ZR), pl.ds(col0, CHW)],
                             wsem)
        lax.fori_loop(0, SLAB // ZR, _wf, None)

        def _wd(j, _):
            pltpu.make_async_copy(
                table.at[pl.ds(tid * SLAB, ZR)],
                sums_hbm.at[pl.ds(tid * SLAB, ZR), pl.ds(col0, CHW)],
                wsem).wait()
        lax.fori_loop(0, SLAB // ZR, _wd, None)
    lax.fori_loop(0, NPASS, _pass, None)

    _zero_slab()

    def _ckey(m, _):
        k16 = keys_v[m // 5, pl.ds((m % 5) * L, L)]
        lk = k16 - cid * KHALF
        ok = (lk >= 0) & (lk < KHALF)
        keys_v[m // 5, pl.ds((m % 5) * L, L)] = jnp.where(ok, lk, KHALF)
    lax.fori_loop(0, nb * (BS // L), _ckey, None)
    plsc.subcore_barrier()

    def _cfire(i, _):
        pltpu.async_copy(obuf, table.at[keys_v.at[i]], ssem, add=True)
    lax.fori_loop(0, nb, _cfire, None)

    def _cdrain(i, _):
        pltpu.make_async_copy(obuf, table.at[keys_v.at[0]], ssem).wait()
    lax.fori_loop(0, nb, _cdrain, None)
    plsc.subcore_barrier()
    pltpu.sync_copy(table.at[pl.ds(tid * CSLAB, CSLAB)],
                    counts_hbm.at[pl.ds(cid * KHALF + tid * CSLAB, CSLAB), :])


@functools.cache
def _sc_scatter():
    return pl.kernel(
        _sc_scatter_body,
        out_type=(jax.ShapeDtypeStruct((K, C), jnp.bfloat16),
                  jax.ShapeDtypeStruct((K, CHW), jnp.bfloat16),
                  jax.ShapeDtypeStruct((KPAD, BS), jnp.int32)),
        mesh=plsc.VectorSubcoreMesh(core_axis_name="c", subcore_axis_name="s",
                                    num_cores=NC, num_subcores=NS),
        compiler_params=pltpu.CompilerParams(use_tc_tiling_on_sc=False, needs_layout_passes=False),
        scratch_types=[
            pltpu.VMEM_SHARED((K, CHW), jnp.bfloat16),
            pltpu.VMEM((GROWS, CHW), jnp.float32),
            pltpu.VMEM((GROWS, CHW), jnp.float32),
            pltpu.VMEM((GROWS, CHW), jnp.bfloat16),
            pltpu.VMEM((GROWS, 4), jnp.int32),
            pltpu.VMEM((BS, CHW), jnp.bfloat16),
            pltpu.VMEM((ZR, CHW), jnp.bfloat16),
            pltpu.VMEM((TBPC, BS), jnp.int32),
            pltpu.SemaphoreType.DMA,
            pltpu.SemaphoreType.DMA,
            pltpu.SemaphoreType.DMA,
        ],
    )


def _p1_body(sums_ref, counts_ref, w1_ref, b1_ref, g1_ref, beta1_ref,
             y1_ref, scale_ref, shift_ref, s_acc, q_acc, m_acc):
    i = pl.program_id(0)

    @pl.when(i == 0)
    def _():
        s_acc[...] = jnp.zeros_like(s_acc)
        q_acc[...] = jnp.zeros_like(q_acc)
        m_acc[...] = jnp.zeros_like(m_acc)

    cnt = counts_ref[:, 0:1].astype(jnp.float32)
    valid = cnt > 0.0
    maskf = valid.astype(jnp.float32)
    inv = jnp.where(valid, 1.0 / jnp.maximum(cnt, 1.0), 0.0)
    x = (sums_ref[...].astype(jnp.float32) * inv).astype(jnp.bfloat16)
    y = lax.dot_general(x, w1_ref[...].astype(jnp.bfloat16),
                        (((1,), (1,)), ((), ())),
                        preferred_element_type=jnp.float32) + b1_ref[...]
    y1_ref[...] = y.astype(jnp.bfloat16)
    ym = y * maskf
    s_acc[...] += jnp.sum(ym, axis=0, keepdims=True)
    q_acc[...] += jnp.sum(y * ym, axis=0, keepdims=True)
    m_acc[...] += jnp.sum(maskf).reshape(1, 1)

    @pl.when(i == GRID - 1)
    def _():
        m = m_acc[0, 0]
        mu = s_acc[...] / m
        var = q_acc[...] / m - mu * mu
        sc = g1_ref[...] * lax.rsqrt(var + EPSV)
        scale_ref[...] = sc
        shift_ref[...] = beta1_ref[...] - mu * sc


def _tc_p1(sums, counts, W1, b1, g1, beta1):
    return pl.pallas_call(
        _p1_body,
        grid=(GRID,),
        in_specs=[
            pl.BlockSpec((RB, C), lambda i: (i, 0)),
            pl.BlockSpec((RB, CHW), lambda i: (i, 0)),
            pl.BlockSpec((C, C), lambda i: (0, 0)),
            pl.BlockSpec((1, C), lambda i: (0, 0)),
            pl.BlockSpec((1, C), lambda i: (0, 0)),
            pl.BlockSpec((1, C), lambda i: (0, 0)),
        ],
        out_specs=[
            pl.BlockSpec((RB, C), lambda i: (i, 0)),
            pl.BlockSpec((1, C), lambda i: (0, 0)),
            pl.BlockSpec((1, C), lambda i: (0, 0)),
        ],
        out_shape=[
            jax.ShapeDtypeStruct((K, C), jnp.bfloat16),
            jax.ShapeDtypeStruct((1, C), jnp.float32),
            jax.ShapeDtypeStruct((1, C), jnp.float32),
        ],
        scratch_shapes=[
            pltpu.VMEM((1, C), jnp.float32),
            pltpu.VMEM((1, C), jnp.float32),
            pltpu.VMEM((1, 1), jnp.float32),
        ],
    )(sums, counts, W1, b1.reshape(1, C), g1.reshape(1, C),
      beta1.reshape(1, C))


def _p2_body(y1_ref, counts_ref, sc1_ref, sh1_ref, w2_ref, b2_ref, g2_ref,
             beta2_ref, y2_ref, scale_ref, shift_ref, s_acc, q_acc, m_acc):
    i = pl.program_id(0)

    @pl.when(i == 0)
    def _():
        s_acc[...] = jnp.zeros_like(s_acc)
        q_acc[...] = jnp.zeros_like(q_acc)
        m_acc[...] = jnp.zeros_like(m_acc)

    cnt = counts_ref[:, 0:1].astype(jnp.float32)
    maskf = (cnt > 0.0).astype(jnp.float32)
    h = jnp.maximum(y1_ref[...].astype(jnp.float32) * sc1_ref[...]
                    + sh1_ref[...], 0.0)
    y = lax.dot_general(h.astype(jnp.bfloat16),
                        w2_ref[...].astype(jnp.bfloat16),
                        (((1,), (1,)), ((), ())),
                        preferred_element_type=jnp.float32) + b2_ref[...]
    y2_ref[...] = y
    ym = y * maskf
    s_acc[...] += jnp.sum(ym, axis=0, keepdims=True)
    q_acc[...] += jnp.sum(y * ym, axis=0, keepdims=True)
    m_acc[...] += jnp.sum(maskf).reshape(1, 1)

    @pl.when(i == GRID - 1)
    def _():
        m = m_acc[0, 0]
        mu = s_acc[...] / m
        var = q_acc[...] / m - mu * mu
        sc = g2_ref[...] * lax.rsqrt(var + EPSV)
        scale_ref[...] = sc
        shift_ref[...] = beta2_ref[...] - mu * sc


def _tc_p2(y1, counts, sc1, sh1, W2, b2, g2, beta2):
    return pl.pallas_call(
        _p2_body,
        grid=(GRID,),
        in_specs=[
            pl.BlockSpec((RB, C), lambda i: (i, 0)),
            pl.BlockSpec((RB, CHW), lambda i: (i, 0)),
            pl.BlockSpec((1, C), lambda i: (0, 0)),
            pl.BlockSpec((1, C), lambda i: (0, 0)),
            pl.BlockSpec((C, C), lambda i: (0, 0)),
            pl.BlockSpec((1, C), lambda i: (0, 0)),
            pl.BlockSpec((1, C), lambda i: (0, 0)),
            pl.BlockSpec((1, C), lambda i: (0, 0)),
        ],
        out_specs=[
            pl.BlockSpec((RB, C), lambda i: (i, 0)),
            pl.BlockSpec((1, C), lambda i: (0, 0)),
            pl.BlockSpec((1, C), lambda i: (0, 0)),
        ],
        out_shape=[
            jax.ShapeDtypeStruct((K, C), jnp.float32),
            jax.ShapeDtypeStruct((1, C), jnp.float32),
            jax.ShapeDtypeStruct((1, C), jnp.float32),
        ],
        scratch_shapes=[
            pltpu.VMEM((1, C), jnp.float32),
            pltpu.VMEM((1, C), jnp.float32),
            pltpu.VMEM((1, 1), jnp.float32),
        ],
    )(y1, counts, sc1, sh1, W2, b2.reshape(1, C), g2.reshape(1, C),
      beta2.reshape(1, C))


GWB = 40


def _sc_gather_body(keys_hbm, feats_hbm, y2_hbm, sc_hbm, sh_hbm, out_hbm,
                    kbuf, fb0, fb1, rw0, rw1, scale_v, shift_v,
                    fsem, gsem, wsem):
    cid = lax.axis_index("c")
    sid = lax.axis_index("s")
    w = sid * NC + cid
    blk0 = w * GWB
    nb = jnp.minimum(NBLK - blk0, GWB)
    pltpu.sync_copy(sc_hbm, scale_v)
    pltpu.sync_copy(sh_hbm, shift_v)
    pltpu.sync_copy(keys_hbm.at[pl.ds(blk0, GWB), :], kbuf)
    svals = [scale_v[pl.ds(j * L, L)] for j in range(C // L)]
    tvals = [shift_v[pl.ds(j * L, L)] for j in range(C // L)]
    fbs, rws = (fb0, fb1), (rw0, rw1)

    def _feats_cp(i, buf):
        return pltpu.make_async_copy(
            feats_hbm.at[pl.ds((blk0 + i) * B, B), :], buf, fsem)

    def _gath_cp(i, buf):
        return pltpu.make_async_copy(y2_hbm.at[kbuf.at[i]], buf, gsem)

    def _out_cp(i, buf):
        return pltpu.make_async_copy(
            buf, out_hbm.at[pl.ds((blk0 + i) * B, B), :], wsem)

    _feats_cp(0, fb0).start()
    _gath_cp(0, rw0).start()

    def _outer(io, _):
        for par in range(2):
            i = io * 2 + par
            fb, rw = fbs[par], rws[par]
            fbn, rwn = fbs[1 - par], rws[1 - par]

            @pl.when(i < nb)
            def _():
                _feats_cp(i, fb).wait()
                _gath_cp(i, rw).wait()

                @pl.when(i + 1 < nb)
                def _():
                    @pl.when(i >= 1)
                    def _():
                        _out_cp(i - 1, fbn).wait()
                    _feats_cp(i + 1, fbn).start()
                    _gath_cp(i + 1, rwn).start()

                def _row(r, _):
                    for j in range(C // L):
                        y = rw[r, pl.ds(j * L, L)]
                        h = jnp.maximum(y * svals[j] + tvals[j], 0.0)
                        fb[r, pl.ds(j * L, L)] += h
                lax.fori_loop(0, B, _row, None)
                _out_cp(i, fb).start()
    lax.fori_loop(0, GWB // 2, _outer, None)
    _out_cp(nb - 2, fb0).wait()
    _out_cp(nb - 1, fb1).wait()


@functools.cache
def _sc_gather():
    return pl.kernel(
        _sc_gather_body,
        out_type=jax.ShapeDtypeStruct((N, C), jnp.float32),
        mesh=plsc.VectorSubcoreMesh(core_axis_name="c", subcore_axis_name="s",
                                    num_cores=NC, num_subcores=NS),
        scratch_types=[
            pltpu.VMEM((GWB, BS), jnp.int32),
            pltpu.VMEM((B, C), jnp.float32),
            pltpu.VMEM((B, C), jnp.float32),
            pltpu.VMEM((B, C), jnp.float32),
            pltpu.VMEM((B, C), jnp.float32),
            pltpu.VMEM((C,), jnp.float32),
            pltpu.VMEM((C,), jnp.float32),
            pltpu.SemaphoreType.DMA,
            pltpu.SemaphoreType.DMA,
            pltpu.SemaphoreType.DMA,
        ],
    )


def kernel(coords, feats, W1, b1, g1, beta1, W2, b2, g2, beta2):
    sums, counts, keys = _sc_scatter()(coords, feats)
    y1, sc1, sh1 = _tc_p1(sums, counts, W1, b1, g1, beta1)
    y2, sc2, sh2 = _tc_p2(y1, counts, sc1, sh1, W2, b2, g2, beta2)
    return _sc_gather()(keys, feats, y2, sc2.reshape(C), sh2.reshape(C))
